# Initial kernel scaffold; baseline (speedup 1.0000x reference)
import jax
import jax.numpy as jnp
from jax.experimental import pallas as pl


def kernel(timesteps):
    raise NotImplementedError("write your pallas kernel here")


# speedup vs baseline: 1.8574x; 1.8574x over previous
import functools
import math

import jax
import jax.numpy as jnp
from jax import lax
from jax.experimental import pallas as pl
from jax.experimental.pallas import tpu as pltpu
from jax.experimental.pallas import tpu_sc as plsc

EMBEDDING_DIM = 128
MAX_LEN = 8192
BATCH = 16384

_info = plsc.get_sparse_core_info()
_NC, _NS = _info.num_cores, _info.num_subcores
_NW = _NC * _NS
_BPW = BATCH // _NW


def _pe_table() -> jnp.ndarray:
    position = jnp.arange(MAX_LEN, dtype=jnp.float32).reshape(-1, 1)
    div_term = jnp.exp(
        jnp.arange(0, EMBEDDING_DIM, 2, dtype=jnp.float32)
        * (-math.log(10000.0) / EMBEDDING_DIM)
    )
    ang = position * div_term
    return jnp.stack([jnp.sin(ang), jnp.cos(ang)], axis=-1).reshape(
        MAX_LEN, EMBEDDING_DIM
    )


@functools.partial(
    pl.kernel,
    mesh=plsc.VectorSubcoreMesh(core_axis_name="c", subcore_axis_name="s"),
    out_type=jax.ShapeDtypeStruct((BATCH, EMBEDDING_DIM), jnp.float32),
    scratch_types=[
        pltpu.VMEM((_BPW,), jnp.int32),
        pltpu.VMEM((_BPW, EMBEDDING_DIM), jnp.float32),
        pltpu.SemaphoreType.DMA,
    ],
)
def _gather(table_hbm, idx_hbm, out_hbm, idx_v, rows_v, sem):
    wid = lax.axis_index("s") * _NC + lax.axis_index("c")
    base = wid * _BPW
    pltpu.sync_copy(idx_hbm.at[pl.ds(base, _BPW)], idx_v)
    pltpu.async_copy(table_hbm.at[idx_v], rows_v, sem).wait()
    pltpu.sync_copy(rows_v, out_hbm.at[pl.ds(base, _BPW)])


def kernel(timesteps):
    table = _pe_table()
    return _gather(table, timesteps.astype(jnp.int32))
